# baseline (device time: 128692 ns/iter reference)
import jax
import jax.numpy as jnp
from jax import lax
from jax.experimental import pallas as pl
from jax.experimental.pallas import tpu as pltpu

N_DEV = 4
M_PER = 1024
Q = 256
K = 4096
N_PER = 2048


def kernel(x, w_mat, scale_x, scale_w):
    my = lax.axis_index("i")
    w_my = lax.dynamic_slice_in_dim(w_mat, my * N_PER, N_PER, axis=1)
    scale = (scale_x * scale_w).astype(jnp.float32)

    def body(x_ref, w_ref, scale_ref, out_ref, xfull, stage,
             send_r, recv_r, send_l, recv_l, copy_sems):
        me = lax.axis_index("i")
        left = lax.rem(me + N_DEV - 1, N_DEV)
        right = lax.rem(me + 1, N_DEV)
        opp = lax.rem(me + 2, N_DEV)

        barrier = pltpu.get_barrier_semaphore()
        for nbr in (left, right):
            pl.semaphore_signal(barrier, inc=1, device_id=(nbr,),
                                device_id_type=pl.DeviceIdType.MESH)
        pl.semaphore_wait(barrier, 2)

        def remote(src, dst, ssem, rsem, tgt):
            return pltpu.make_async_remote_copy(
                src_ref=src, dst_ref=dst, send_sem=ssem, recv_sem=rsem,
                device_id=(tgt,), device_id_type=pl.DeviceIdType.MESH)

        r_own = []
        l_own = []
        for q in range(4):
            src = x_ref.at[pl.ds(q * Q, Q), :]
            dst = xfull.at[me, pl.ds(q * Q, Q), :]
            r_own.append(remote(src, dst, send_r.at[q], recv_r.at[q], right))
            l_own.append(remote(src, dst, send_l.at[q], recv_l.at[q], left))
            r_own[q].start()
            l_own[q].start()

        copies = []

        def gemm_store(a_i8, origin, q):
            k = len(copies)
            slot = k % 4
            if k >= 4:
                copies[k - 4].wait()
            acc = lax.dot_general(a_i8, w_ref[...], (((1,), (0,)), ((), ())),
                                  preferred_element_type=jnp.int32)
            y = acc.astype(jnp.float32) * scale_ref[0]
            stage[slot, :, :] = y * jax.nn.sigmoid(y)
            cp = pltpu.make_async_copy(
                stage.at[slot],
                out_ref.at[pl.ds(origin * M_PER + q * Q, Q), :],
                copy_sems.at[slot])
            cp.start()
            copies.append(cp)

        def fwd(origin, q, ssem, rsem, tgt):
            r = remote(xfull.at[origin, pl.ds(q * Q, Q), :],
                       xfull.at[origin, pl.ds(q * Q, Q), :],
                       ssem, rsem, tgt)
            r.start()
            return r

        for q in range(4):
            gemm_store(x_ref[pl.ds(q * Q, Q), :], me, q)

        fwds = []
        for q in range(4):
            r_own[q].wait_recv()
            if q < 2:
                fwds.append(fwd(left, q, send_r.at[4 + q],
                                recv_r.at[4 + q], right))
            gemm_store(xfull[left, pl.ds(q * Q, Q), :], left, q)

            l_own[q].wait_recv()
            if q >= 2:
                fwds.append(fwd(right, q, send_l.at[2 + q],
                                recv_l.at[2 + q], left))
            gemm_store(xfull[right, pl.ds(q * Q, Q), :], right, q)

        fwds[0].wait_recv()
        gemm_store(xfull[opp, pl.ds(0 * Q, Q), :], opp, 0)
        fwds[2].wait_recv()
        gemm_store(xfull[opp, pl.ds(2 * Q, Q), :], opp, 2)
        fwds[1].wait_recv()
        gemm_store(xfull[opp, pl.ds(1 * Q, Q), :], opp, 1)
        fwds[3].wait_recv()
        gemm_store(xfull[opp, pl.ds(3 * Q, Q), :], opp, 3)

        for cp in copies[-4:]:
            cp.wait()
        for rd in (*r_own, *l_own, *fwds):
            rd.wait_send()

    return pl.pallas_call(
        body,
        out_shape=jax.ShapeDtypeStruct((N_DEV * M_PER, N_PER), jnp.float32),
        in_specs=[
            pl.BlockSpec(memory_space=pltpu.VMEM),
            pl.BlockSpec(memory_space=pltpu.VMEM),
            pl.BlockSpec(memory_space=pltpu.SMEM),
        ],
        out_specs=pl.BlockSpec(memory_space=pl.ANY),
        scratch_shapes=[
            pltpu.VMEM((N_DEV, M_PER, K), jnp.int8),
            pltpu.VMEM((4, Q, N_PER), jnp.float32),
            pltpu.SemaphoreType.DMA((6,)),
            pltpu.SemaphoreType.DMA((6,)),
            pltpu.SemaphoreType.DMA((6,)),
            pltpu.SemaphoreType.DMA((6,)),
            pltpu.SemaphoreType.DMA((4,)),
        ],
        compiler_params=pltpu.CompilerParams(
            collective_id=0,
            vmem_limit_bytes=100 * 1024 * 1024,
        ),
    )(x, w_my, scale)


# device time: 113853 ns/iter; 1.1303x vs baseline; 1.1303x over previous
import jax
import jax.numpy as jnp
from jax import lax
from jax.experimental import pallas as pl
from jax.experimental.pallas import tpu as pltpu

N_DEV = 4
M_PER = 1024
Q = 256
K = 4096
N_PER = 2048


def kernel(x, w_mat, scale_x, scale_w):
    scale = (scale_x * scale_w).astype(jnp.float32)

    def body(x_ref, w_hbm, scale_ref, out_ref, xfull, w_ref, stage,
             send_r, recv_r, send_l, recv_l, w_sem, copy_sems):
        me = lax.axis_index("i")
        left = lax.rem(me + N_DEV - 1, N_DEV)
        right = lax.rem(me + 1, N_DEV)
        opp = lax.rem(me + 2, N_DEV)

        cp_w = pltpu.make_async_copy(
            w_hbm.at[:, pl.ds(me * N_PER, N_PER)], w_ref, w_sem)
        cp_w.start()

        barrier = pltpu.get_barrier_semaphore()
        for nbr in (left, right):
            pl.semaphore_signal(barrier, inc=1, device_id=(nbr,),
                                device_id_type=pl.DeviceIdType.MESH)
        pl.semaphore_wait(barrier, 2)

        def remote(src, dst, ssem, rsem, tgt):
            return pltpu.make_async_remote_copy(
                src_ref=src, dst_ref=dst, send_sem=ssem, recv_sem=rsem,
                device_id=(tgt,), device_id_type=pl.DeviceIdType.MESH)

        r_own = []
        l_own = []
        for q in range(4):
            src = x_ref.at[pl.ds(q * Q, Q), :]
            dst = xfull.at[me, pl.ds(q * Q, Q), :]
            r_own.append(remote(src, dst, send_r.at[q], recv_r.at[q], right))
            l_own.append(remote(src, dst, send_l.at[q], recv_l.at[q], left))
            r_own[q].start()
            l_own[q].start()

        copies = []

        def gemm_store(a_i8, origin, q):
            k = len(copies)
            slot = k % 4
            if k >= 4:
                copies[k - 4].wait()
            acc = lax.dot_general(a_i8, w_ref[...], (((1,), (0,)), ((), ())),
                                  preferred_element_type=jnp.int32)
            y = acc.astype(jnp.float32) * scale_ref[0]
            stage[slot, :, :] = y * jax.nn.sigmoid(y)
            cp = pltpu.make_async_copy(
                stage.at[slot],
                out_ref.at[pl.ds(origin * M_PER + q * Q, Q), :],
                copy_sems.at[slot])
            cp.start()
            copies.append(cp)

        def fwd(origin, q, ssem, rsem, tgt):
            r = remote(xfull.at[origin, pl.ds(q * Q, Q), :],
                       xfull.at[origin, pl.ds(q * Q, Q), :],
                       ssem, rsem, tgt)
            r.start()
            return r

        cp_w.wait()
        for q in range(4):
            gemm_store(x_ref[pl.ds(q * Q, Q), :], me, q)

        fwds = []
        for q in range(4):
            r_own[q].wait_recv()
            if q < 2:
                fwds.append(fwd(left, q, send_r.at[4 + q],
                                recv_r.at[4 + q], right))
            gemm_store(xfull[left, pl.ds(q * Q, Q), :], left, q)

            l_own[q].wait_recv()
            if q >= 2:
                fwds.append(fwd(right, q, send_l.at[2 + q],
                                recv_l.at[2 + q], left))
            gemm_store(xfull[right, pl.ds(q * Q, Q), :], right, q)

        fwds[0].wait_recv()
        gemm_store(xfull[opp, pl.ds(0 * Q, Q), :], opp, 0)
        fwds[2].wait_recv()
        gemm_store(xfull[opp, pl.ds(2 * Q, Q), :], opp, 2)
        fwds[1].wait_recv()
        gemm_store(xfull[opp, pl.ds(1 * Q, Q), :], opp, 1)
        fwds[3].wait_recv()
        gemm_store(xfull[opp, pl.ds(3 * Q, Q), :], opp, 3)

        for cp in copies[-4:]:
            cp.wait()
        for rd in (*r_own, *l_own, *fwds):
            rd.wait_send()

    return pl.pallas_call(
        body,
        out_shape=jax.ShapeDtypeStruct((N_DEV * M_PER, N_PER), jnp.float32),
        in_specs=[
            pl.BlockSpec(memory_space=pltpu.VMEM),
            pl.BlockSpec(memory_space=pl.ANY),
            pl.BlockSpec(memory_space=pltpu.SMEM),
        ],
        out_specs=pl.BlockSpec(memory_space=pl.ANY),
        scratch_shapes=[
            pltpu.VMEM((N_DEV, M_PER, K), jnp.int8),
            pltpu.VMEM((K, N_PER), jnp.int8),
            pltpu.VMEM((4, Q, N_PER), jnp.float32),
            pltpu.SemaphoreType.DMA((6,)),
            pltpu.SemaphoreType.DMA((6,)),
            pltpu.SemaphoreType.DMA((6,)),
            pltpu.SemaphoreType.DMA((6,)),
            pltpu.SemaphoreType.DMA,
            pltpu.SemaphoreType.DMA((4,)),
        ],
        compiler_params=pltpu.CompilerParams(
            collective_id=0,
            vmem_limit_bytes=100 * 1024 * 1024,
        ),
    )(x, w_mat, scale)
